# baseline (speedup 1.0000x reference)
import functools

import jax
import jax.numpy as jnp
from jax import lax
from jax.experimental import pallas as pl
from jax.experimental.pallas import tpu as pltpu
from jax.experimental.pallas import tpu_sc as plsc

U_NUM = 1000000
I_NUM = 100000
DIM = 16
B = 16384

_NC = 2
_NS = 16
_NW = _NC * _NS
_BPW = B // _NW
_CHUNK = 128
_NPASS = _BPW // _CHUNK
_NBUF = 2

_mesh = plsc.VectorSubcoreMesh(core_axis_name="c", subcore_axis_name="s")


@functools.partial(
    pl.kernel,
    out_type=jax.ShapeDtypeStruct((B,), jnp.float32),
    mesh=_mesh,
    scratch_types=[
        pltpu.VMEM((_BPW,), jnp.int32),
        pltpu.VMEM((_BPW,), jnp.int32),
        pltpu.VMEM((_BPW,), jnp.int32),
        pltpu.VMEM((_BPW,), jnp.int32),
        pltpu.VMEM((_BPW,), jnp.int32),
        pltpu.VMEM((_NBUF, _CHUNK, 128), jnp.float32),
        pltpu.VMEM((_NBUF, _CHUNK, 128), jnp.float32),
        pltpu.VMEM((_NBUF, _CHUNK, 128), jnp.float32),
        pltpu.VMEM((_BPW, DIM), jnp.float32),
        pltpu.VMEM((_BPW,), jnp.float32),
        pltpu.SemaphoreType.DMA,
        pltpu.SemaphoreType.DMA,
        pltpu.SemaphoreType.DMA,
    ],
    compiler_params=pltpu.CompilerParams(
        needs_layout_passes=False, use_tc_tiling_on_sc=False
    ),
)
def _irt_sc(uid_hbm, iid_hbm, theta_hbm, a_hbm, b_hbm, c_hbm, out_hbm,
            uid_v, iid_v, urow_v, irow_v, crow_v, th_v, a_v, b_v, c_v,
            out_v, sem0, sem1, semc):
    sems = (sem0, sem1)
    wid = lax.axis_index("s") * _NC + lax.axis_index("c")
    base = wid * _BPW

    pltpu.sync_copy(uid_hbm.at[pl.ds(base, _BPW)], uid_v)
    pltpu.sync_copy(iid_hbm.at[pl.ds(base, _BPW)], iid_v)

    def idx_body(j, _):
        s = pl.ds(j * 16, 16)
        u = uid_v[s]
        i = iid_v[s]
        urow_v[s] = lax.shift_right_logical(u, 3)
        irow_v[s] = lax.shift_right_logical(i, 3)
        crow_v[s] = lax.shift_right_logical(i, 4)
        return _
    lax.fori_loop(0, _BPW // 16, idx_body, 0, unroll=False)

    def fire(p):
        sl = pl.ds(p * _CHUNK, _CHUNK)
        buf = p % _NBUF
        s = sems[buf]
        return (
            pltpu.async_copy(theta_hbm.at[urow_v.at[sl]], th_v.at[buf], s),
            pltpu.async_copy(a_hbm.at[irow_v.at[sl]], a_v.at[buf], s),
            pltpu.async_copy(b_hbm.at[irow_v.at[sl]], b_v.at[buf], s),
        )

    ccps = []
    for k in range(_NPASS):
        sl = pl.ds(k * _CHUNK, _CHUNK)
        ccps.append(pltpu.async_copy(c_hbm.at[crow_v.at[sl]], c_v.at[sl, :], semc))

    lane = lax.iota(jnp.int32, 16)
    dcoef = jnp.full((16,), 1.702, jnp.float32)
    one = jnp.full((16,), 1.0, jnp.float32)

    def compute_pass(p):
        buf = p % _NBUF
        thp = th_v.at[buf]
        ap = a_v.at[buf]
        bp = b_v.at[buf]
        for blk in range(_CHUNK // 16):
            g = p * _CHUNK + blk * 16
            sl16 = pl.ds(g, 16)
            u16 = uid_v[sl16]
            i16 = iid_v[sl16]
            ucol = lax.shift_left(u16 & 7, 4)
            icol = lax.shift_left(i16 & 7, 4)
            rows = lane + blk * 16
            acc = jnp.zeros((16,), jnp.float32)
            for t in range(DIM):
                d_idx = (lane + t) & 15
                th = plsc.load_gather(thp, [rows, ucol + d_idx])
                av = plsc.load_gather(ap, [rows, icol + d_idx])
                bv = plsc.load_gather(bp, [rows, icol + d_idx])
                acc = acc + av * (th - bv)
            craw = plsc.load_gather(c_v, [lane + g, i16 & 15])
            cs = one / (one + jnp.exp(-craw))
            sig = one / (one + jnp.exp(-dcoef * acc))
            out_v[sl16] = cs + (one - cs) * sig

    inflight = [fire(0)]
    for p in range(_NPASS):
        if p + 1 < _NPASS:
            inflight.append(fire(p + 1))
        for cp in inflight.pop(0):
            cp.wait()
        if p == 0:
            for cp in ccps:
                cp.wait()
        compute_pass(p)

    pltpu.sync_copy(out_v, out_hbm.at[pl.ds(base, _BPW)])


def kernel(user_id, item_id, theta_w, a_w, b_w, c_w):
    uid = jnp.asarray(user_id, jnp.int32)
    iid = jnp.asarray(item_id, jnp.int32)
    th_t = jnp.reshape(theta_w, (U_NUM // 8, 128))
    a_t = jnp.reshape(a_w, (I_NUM // 8, 128))
    b_t = jnp.reshape(b_w, (I_NUM // 8, 128))
    c_t = jnp.reshape(c_w, (I_NUM // DIM, DIM))
    return _irt_sc(uid, iid, th_t, a_t, b_t, c_t)
